# Initial kernel scaffold; baseline (speedup 1.0000x reference)
import jax
import jax.numpy as jnp
from jax.experimental import pallas as pl


def kernel(rules, global_to_local, W):
    raise NotImplementedError("write your pallas kernel here")


# speedup vs baseline: 726.8035x; 726.8035x over previous
import functools

import jax
import jax.numpy as jnp
from jax import lax
from jax.experimental import pallas as pl
from jax.experimental.pallas import tpu as pltpu
from jax.experimental.pallas import tpu_sc as plsc

_PAD = 50000
_B = 16384
_L = 200
_G2L_LEN = 100001
_FLEN = 102400
_W_LEN = 50001
_W_PAD = 50016
_NT = 32
_N1 = _FLEN // _NT
_ROWS = _B // _NT
_CH = 64
_NCH = _ROWS // _CH
_LANES = 16

_mesh = plsc.VectorSubcoreMesh(
    core_axis_name="c", subcore_axis_name="s", num_cores=2, num_subcores=16)


@functools.partial(
    pl.kernel,
    out_type=jax.ShapeDtypeStruct((_FLEN,), jnp.float32),
    mesh=_mesh,
    compiler_params=pltpu.CompilerParams(needs_layout_passes=False),
    scratch_types=[
        pltpu.VMEM((_W_PAD,), jnp.float32),
        pltpu.VMEM((_N1,), jnp.int32),
        pltpu.VMEM((_N1,), jnp.float32),
    ],
)
def _build_f(g2l_hbm, w_hbm, f_hbm, w_v, g2l_v, f_v):
    wid = lax.axis_index("s") * 2 + lax.axis_index("c")
    base = wid * _N1
    pltpu.sync_copy(w_hbm, w_v)
    pltpu.sync_copy(g2l_hbm.at[pl.ds(base, _N1)], g2l_v)

    def body(i, _):
        off = i * _LANES
        g = g2l_v[pl.ds(off, _LANES)]
        w = plsc.load_gather(w_v, [g])
        f = jnp.where(g == _PAD, 1.0, 1.0 / (1.0 + jnp.exp(w)))
        f_v[pl.ds(off, _LANES)] = f
        return 0

    lax.fori_loop(0, _N1 // _LANES, body, 0)
    pltpu.sync_copy(f_v, f_hbm.at[pl.ds(base, _N1)])


@functools.partial(
    pl.kernel,
    out_type=jax.ShapeDtypeStruct((_B,), jnp.float32),
    mesh=_mesh,
    compiler_params=pltpu.CompilerParams(needs_layout_passes=False),
    scratch_types=[
        pltpu.VMEM((_FLEN,), jnp.float32),
        pltpu.VMEM((_CH * _L,), jnp.int32),
        pltpu.VMEM((_CH * _L,), jnp.int32),
        pltpu.VMEM((_ROWS,), jnp.float32),
        pltpu.SemaphoreType.DMA,
        pltpu.SemaphoreType.DMA,
        pltpu.SemaphoreType.DMA,
    ],
)
def _noisy_or(rules_hbm, f_hbm, out_hbm, f_v, rb0, rb1, out_v,
              sem_f, sem0, sem1):
    wid = lax.axis_index("s") * 2 + lax.axis_index("c")
    rbase = wid * (_ROWS * _L)
    rbufs = (rb0, rb1)
    sems = (sem0, sem1)

    cp_f = pltpu.async_copy(f_hbm, f_v, sem_f)
    handles = {}
    for c in range(min(2, _NCH)):
        handles[c] = pltpu.async_copy(
            rules_hbm.at[pl.ds(rbase + c * _CH * _L, _CH * _L)],
            rbufs[c % 2], sems[c % 2])
    cp_f.wait()

    lane = lax.broadcasted_iota(jnp.int32, (_LANES,), 0)
    for c in range(_NCH):
        handles.pop(c).wait()
        rbuf = rbufs[c % 2]
        for g in range(_CH // _LANES):
            base_iv = (lane + g * _LANES) * _L
            accs = [jnp.full((_LANES,), 1.0, jnp.float32) for _ in range(4)]

            def body(j, accs, rbuf=rbuf, base_iv=base_iv):
                out = list(accs)
                for k in range(8):
                    iv = base_iv + (j * 8 + k)
                    rid = plsc.load_gather(rbuf, [iv])
                    fv = plsc.load_gather(f_v, [rid])
                    out[k % 4] = out[k % 4] * fv
                return tuple(out)

            a0, a1, a2, a3 = lax.fori_loop(0, _L // 8, body, tuple(accs))
            prod = (a0 * a1) * (a2 * a3)
            res = jnp.clip(1.0 - prod, 0.0001, 0.99999)
            out_v[pl.ds(c * _CH + g * _LANES, _LANES)] = res
        nxt = c + 2
        if nxt < _NCH:
            handles[nxt] = pltpu.async_copy(
                rules_hbm.at[pl.ds(rbase + nxt * _CH * _L, _CH * _L)],
                rbuf, sems[c % 2])
    pltpu.sync_copy(out_v, out_hbm.at[pl.ds(wid * _ROWS, _ROWS)])


def kernel(rules, global_to_local, W):
    g2l_p = jnp.concatenate([
        global_to_local.astype(jnp.int32),
        jnp.full((_FLEN - _G2L_LEN,), _PAD, jnp.int32),
    ])
    w_pad = jnp.concatenate([
        W.reshape(-1),
        jnp.zeros((_W_PAD - _W_LEN,), jnp.float32),
    ])
    f_table = _build_f(g2l_p, w_pad)
    out = _noisy_or(rules.reshape(-1), f_table)
    return out.reshape(_B, 1)
